# baseline (speedup 1.0000x reference)
import functools

import jax
import jax.numpy as jnp
from jax import lax
from jax.experimental import pallas as pl
from jax.experimental.pallas import tpu as pltpu
from jax.experimental.pallas import tpu_sc as plsc

_N = 10000
_E = 320000
_D = 128
_H = 128
_C = 10
_G = 64

_NC = 2
_NS = 16
_NW = _NC * _NS
_CHUNK = 32
_CPW = 320
_EPW = _CPW * _CHUNK
_EPAD = _NW * _EPW
_NPAD = 10112
_RPT = _NPAD // _NS
_TRASH = _N
_UNROLL = 32
_DEPTH = 8


_NBINS = 10240
_L = 16


def _seg_body(with_deg, depth, *refs):
    if with_deg:
        (h, srcp, dstp, zrow, zflat, agg_out, deg_out,
         src_v, dst_v, *rest) = refs
        rows_b = rest[:depth]
        hist_v, agg_s = rest[depth], rest[depth + 1]
        sem_b = rest[depth + 2:]
    else:
        (h, srcp, dstp, zrow, agg_out,
         src_v, dst_v, *rest) = refs
        rows_b = rest[:depth]
        agg_s = rest[depth]
        sem_b = rest[depth + 1:]
    c = lax.axis_index("c")
    s = lax.axis_index("s")
    wid = s * _NC + c

    rows = pl.ds(s * _RPT, _RPT)
    pltpu.sync_copy(zrow.at[rows], agg_s.at[rows])
    if with_deg:
        pltpu.sync_copy(zflat, hist_v)
    plsc.subcore_barrier()

    @pl.loop(0, _CPW // _UNROLL)
    def _chunks(step):
        seg = pl.ds(step * _UNROLL, _UNROLL)
        pltpu.sync_copy(srcp.at[wid, seg], src_v)
        pltpu.sync_copy(dstp.at[wid, seg], dst_v)
        cps = [None] * depth
        for d in range(depth - 1):
            cps[d] = pltpu.async_copy(h.at[src_v.at[d]], rows_b[d], sem_b[d])
        for i in range(_UNROLL):
            nxt = i + depth - 1
            if nxt < _UNROLL:
                cps[nxt % depth] = pltpu.async_copy(
                    h.at[src_v.at[nxt]], rows_b[nxt % depth],
                    sem_b[nxt % depth])
            cps[i % depth].wait()
            pltpu.sync_copy(rows_b[i % depth], agg_s.at[dst_v.at[i]],
                            add=True)
            if with_deg:
                for k in range(_CHUNK // _L):
                    d = dst_v[i, pl.ds(k * _L, _L)]
                    plsc.addupdate_scatter(
                        hist_v, [d], jnp.ones((_L,), jnp.float32))

    plsc.subcore_barrier()
    pltpu.sync_copy(agg_s.at[rows], agg_out.at[c, rows])
    if with_deg:
        pltpu.sync_copy(hist_v, deg_out.at[c, s])


def _make_seg(with_deg, depth):
    mesh = plsc.VectorSubcoreMesh(core_axis_name="c", subcore_axis_name="s",
                                  num_cores=_NC, num_subcores=_NS)
    out_type = [jax.ShapeDtypeStruct((_NC, _NPAD, _H), jnp.float32)]
    scratch = [
        pltpu.VMEM((_UNROLL, _CHUNK), jnp.int32),
        pltpu.VMEM((_UNROLL, _CHUNK), jnp.int32),
    ]
    for _ in range(depth):
        scratch.append(pltpu.VMEM((_CHUNK, _H), jnp.float32))
    if with_deg:
        out_type.append(
            jax.ShapeDtypeStruct((_NC, _NS, _NBINS), jnp.float32))
        scratch.append(pltpu.VMEM((_NBINS,), jnp.float32))
    scratch.append(pltpu.VMEM_SHARED((_NPAD, _H), jnp.float32))
    for _ in range(depth):
        scratch.append(pltpu.SemaphoreType.DMA)
    return pl.kernel(
        functools.partial(_seg_body, with_deg, depth),
        out_type=tuple(out_type) if with_deg else out_type[0],
        mesh=mesh,
        scratch_types=tuple(scratch),
        compiler_params=pltpu.CompilerParams(needs_layout_passes=False),
    )


_seg1 = _make_seg(True, 6)
_seg = _make_seg(False, _DEPTH)

_BLK = 512
_NGRID = (_N + _BLK - 1) // _BLK


def _layer1_body(agg_ref, deg_ref, h_ref, wl_ref, bl_ref, wr_ref,
                 out_ref, dinv_ref):
    deg = jnp.sum(deg_ref[...], axis=0)
    dinv = 1.0 / jnp.maximum(deg, 1.0)
    dinv_ref[...] = dinv
    mean = (agg_ref[0] + agg_ref[1]) * dinv
    h = h_ref[...]
    z = (jnp.dot(mean, wl_ref[...], preferred_element_type=jnp.float32, precision=lax.Precision.HIGHEST)
         + bl_ref[...]
         + jnp.dot(h, wr_ref[...], preferred_element_type=jnp.float32, precision=lax.Precision.HIGHEST))
    out_ref[...] = jnp.maximum(z, 0.0) + h


_tc_layer1 = pl.pallas_call(
    _layer1_body,
    grid=(_NGRID,),
    in_specs=[
        pl.BlockSpec((_NC, _BLK, _H), lambda i: (0, i, 0)),
        pl.BlockSpec((_NW, _BLK, 1), lambda i: (0, i, 0)),
        pl.BlockSpec((_BLK, _H), lambda i: (i, 0)),
        pl.BlockSpec((_H, _H), lambda i: (0, 0)),
        pl.BlockSpec((1, _H), lambda i: (0, 0)),
        pl.BlockSpec((_H, _H), lambda i: (0, 0)),
    ],
    out_specs=[
        pl.BlockSpec((_BLK, _H), lambda i: (i, 0)),
        pl.BlockSpec((_BLK, 1), lambda i: (i, 0)),
    ],
    out_shape=[
        jax.ShapeDtypeStruct((_N, _H), jnp.float32),
        jax.ShapeDtypeStruct((_N, 1), jnp.float32),
    ],
)


def _layer_body(agg_ref, dinv_ref, h_ref, wl_ref, bl_ref, wr_ref, out_ref):
    mean = (agg_ref[0] + agg_ref[1]) * dinv_ref[...]
    h = h_ref[...]
    z = (jnp.dot(mean, wl_ref[...], preferred_element_type=jnp.float32, precision=lax.Precision.HIGHEST)
         + bl_ref[...]
         + jnp.dot(h, wr_ref[...], preferred_element_type=jnp.float32, precision=lax.Precision.HIGHEST))
    out_ref[...] = jnp.maximum(z, 0.0) + h


_tc_layer = pl.pallas_call(
    _layer_body,
    grid=(_NGRID,),
    in_specs=[
        pl.BlockSpec((_NC, _BLK, _H), lambda i: (0, i, 0)),
        pl.BlockSpec((_BLK, 1), lambda i: (i, 0)),
        pl.BlockSpec((_BLK, _H), lambda i: (i, 0)),
        pl.BlockSpec((_H, _H), lambda i: (0, 0)),
        pl.BlockSpec((1, _H), lambda i: (0, 0)),
        pl.BlockSpec((_H, _H), lambda i: (0, 0)),
    ],
    out_specs=pl.BlockSpec((_BLK, _H), lambda i: (i, 0)),
    out_shape=jax.ShapeDtypeStruct((_N, _H), jnp.float32),
)


def _final_body(agg_ref, dinv_ref, h_ref, wl_ref, bl_ref, wr_ref,
                batch_ref, wh_ref, bh_ref, out_ref, gsum_ref, gcnt_ref):
    i = pl.program_id(0)

    @pl.when(i == 0)
    def _():
        gsum_ref[...] = jnp.zeros_like(gsum_ref)
        gcnt_ref[...] = jnp.zeros_like(gcnt_ref)

    mean = (agg_ref[0] + agg_ref[1]) * dinv_ref[...]
    h = h_ref[...]
    z = (jnp.dot(mean, wl_ref[...], preferred_element_type=jnp.float32, precision=lax.Precision.HIGHEST)
         + bl_ref[...]
         + jnp.dot(h, wr_ref[...], preferred_element_type=jnp.float32, precision=lax.Precision.HIGHEST))
    h3 = jnp.maximum(z, 0.0) + h
    rowc = i * _BLK + lax.broadcasted_iota(jnp.int32, (_BLK, 1), 0)
    h3 = jnp.where(rowc < _N, h3, 0.0)

    row = i * _BLK + lax.broadcasted_iota(jnp.int32, (_BLK, _G), 0)
    valid = (row < _N).astype(jnp.float32)
    onehot = (batch_ref[...] ==
              lax.broadcasted_iota(jnp.int32, (_BLK, _G), 1)).astype(
                  jnp.float32) * valid
    gsum_ref[...] += lax.dot_general(
        onehot, h3, (((0,), (0,)), ((), ())),
        preferred_element_type=jnp.float32, precision=lax.Precision.HIGHEST)
    gcnt_ref[...] += lax.dot_general(
        onehot, jnp.ones((_BLK, 1), jnp.float32), (((0,), (0,)), ((), ())),
        preferred_element_type=jnp.float32, precision=lax.Precision.HIGHEST)

    @pl.when(i == _NGRID - 1)
    def _():
        g = gsum_ref[...] / jnp.maximum(gcnt_ref[...], 1.0)
        out_ref[...] = (
            jnp.dot(g, wh_ref[...], preferred_element_type=jnp.float32, precision=lax.Precision.HIGHEST)
            + bh_ref[...])


_tc_final = pl.pallas_call(
    _final_body,
    grid=(_NGRID,),
    in_specs=[
        pl.BlockSpec((_NC, _BLK, _H), lambda i: (0, i, 0)),
        pl.BlockSpec((_BLK, 1), lambda i: (i, 0)),
        pl.BlockSpec((_BLK, _H), lambda i: (i, 0)),
        pl.BlockSpec((_H, _H), lambda i: (0, 0)),
        pl.BlockSpec((1, _H), lambda i: (0, 0)),
        pl.BlockSpec((_H, _H), lambda i: (0, 0)),
        pl.BlockSpec((_BLK, 1), lambda i: (i, 0)),
        pl.BlockSpec((_H, _C), lambda i: (0, 0)),
        pl.BlockSpec((1, _C), lambda i: (0, 0)),
    ],
    out_specs=pl.BlockSpec((_G, _C), lambda i: (0, 0)),
    out_shape=jax.ShapeDtypeStruct((_G, _C), jnp.float32),
    scratch_shapes=[
        pltpu.VMEM((_G, _H), jnp.float32),
        pltpu.VMEM((_G, 1), jnp.float32),
    ],
)


def kernel(x, edge_index, batch, W_l1, b_l1, W_r1, W_l2, b_l2, W_r2,
           W_l3, b_l3, W_r3, W_head, b_head):
    src = edge_index[0]
    dst = edge_index[1]
    npad = _EPAD - _E
    srcp = jnp.concatenate(
        [src, jnp.zeros((npad,), jnp.int32)]).reshape(_NW, _CPW, _CHUNK)
    dstp = jnp.concatenate(
        [dst, jnp.full((npad,), _TRASH, jnp.int32)]).reshape(_NW, _CPW, _CHUNK)
    zrow = jnp.zeros((_NPAD, _H), jnp.float32)
    zflat = jnp.zeros((_NBINS,), jnp.float32)
    batch2 = batch.reshape(_N, 1)

    agg1, deg1 = _seg1(x, srcp, dstp, zrow, zflat)
    deg1r = deg1.reshape(_NW, _NBINS, 1)
    h1, dinv = _tc_layer1(agg1, deg1r, x,
                          W_l1.T, b_l1.reshape(1, _H), W_r1.T)
    agg2 = _seg(h1, srcp, dstp, zrow)
    h2 = _tc_layer(agg2, dinv, h1, W_l2.T, b_l2.reshape(1, _H), W_r2.T)
    agg3 = _seg(h2, srcp, dstp, zrow)
    return _tc_final(agg3, dinv, h2, W_l3.T, b_l3.reshape(1, _H), W_r3.T,
                     batch2, W_head.T, b_head.reshape(1, _C))


# speedup vs baseline: 1.0799x; 1.0799x over previous
import functools

import jax
import jax.numpy as jnp
from jax import lax
from jax.experimental import pallas as pl
from jax.experimental.pallas import tpu as pltpu
from jax.experimental.pallas import tpu_sc as plsc

_N = 10000
_E = 320000
_D = 128
_H = 128
_C = 10
_G = 64

_NC = 2
_NS = 16
_NW = _NC * _NS
_CHUNK = 64
_CPW = 160
_EPW = _CPW * _CHUNK
_EPAD = _NW * _EPW
_NPAD = 10112
_RPT = _NPAD // _NS
_TRASH = _N
_UNROLL = 32
_DEPTH = 4


_NBINS = 10240
_L = 16


def _seg_body(with_deg, depth, unroll, *refs):
    if with_deg:
        (h, srcp, dstp, zrow, zflat, agg_out, deg_out,
         src_v, dst_v, *rest) = refs
        rows_b = rest[:depth]
        hist_v, agg_s = rest[depth], rest[depth + 1]
        sem_b = rest[depth + 2:]
    else:
        (h, srcp, dstp, zrow, agg_out,
         src_v, dst_v, *rest) = refs
        rows_b = rest[:depth]
        agg_s = rest[depth]
        sem_b = rest[depth + 1:]
    c = lax.axis_index("c")
    s = lax.axis_index("s")
    wid = s * _NC + c

    rows = pl.ds(s * _RPT, _RPT)
    pltpu.sync_copy(zrow.at[rows], agg_s.at[rows])
    if with_deg:
        pltpu.sync_copy(zflat, hist_v)
    plsc.subcore_barrier()

    @pl.loop(0, _CPW // unroll)
    def _chunks(step):
        seg = pl.ds(step * unroll, unroll)
        pltpu.sync_copy(srcp.at[wid, seg], src_v)
        pltpu.sync_copy(dstp.at[wid, seg], dst_v)
        cps = [None] * depth
        for d in range(depth - 1):
            cps[d] = pltpu.async_copy(h.at[src_v.at[d]], rows_b[d], sem_b[d])
        for i in range(unroll):
            nxt = i + depth - 1
            if nxt < unroll:
                cps[nxt % depth] = pltpu.async_copy(
                    h.at[src_v.at[nxt]], rows_b[nxt % depth],
                    sem_b[nxt % depth])
            cps[i % depth].wait()
            pltpu.sync_copy(rows_b[i % depth], agg_s.at[dst_v.at[i]],
                            add=True)
            if with_deg:
                for k in range(_CHUNK // _L):
                    d = dst_v[i, pl.ds(k * _L, _L)]
                    plsc.addupdate_scatter(
                        hist_v, [d], jnp.ones((_L,), jnp.float32))

    plsc.subcore_barrier()
    pltpu.sync_copy(agg_s.at[rows], agg_out.at[c, rows])
    if with_deg:
        pltpu.sync_copy(hist_v, deg_out.at[c, s])


def _make_seg(with_deg, depth, unroll):
    mesh = plsc.VectorSubcoreMesh(core_axis_name="c", subcore_axis_name="s",
                                  num_cores=_NC, num_subcores=_NS)
    out_type = [jax.ShapeDtypeStruct((_NC, _NPAD, _H), jnp.float32)]
    scratch = [
        pltpu.VMEM((unroll, _CHUNK), jnp.int32),
        pltpu.VMEM((unroll, _CHUNK), jnp.int32),
    ]
    for _ in range(depth):
        scratch.append(pltpu.VMEM((_CHUNK, _H), jnp.float32))
    if with_deg:
        out_type.append(
            jax.ShapeDtypeStruct((_NC, _NS, _NBINS), jnp.float32))
        scratch.append(pltpu.VMEM((_NBINS,), jnp.float32))
    scratch.append(pltpu.VMEM_SHARED((_NPAD, _H), jnp.float32))
    for _ in range(depth):
        scratch.append(pltpu.SemaphoreType.DMA)
    return pl.kernel(
        functools.partial(_seg_body, with_deg, depth, unroll),
        out_type=tuple(out_type) if with_deg else out_type[0],
        mesh=mesh,
        scratch_types=tuple(scratch),
        compiler_params=pltpu.CompilerParams(needs_layout_passes=False),
    )


_seg1 = _make_seg(True, 4, 16)
_seg = _make_seg(False, _DEPTH, _UNROLL)

_BLK = 512
_NGRID = (_N + _BLK - 1) // _BLK


def _layer1_body(agg_ref, deg_ref, h_ref, wl_ref, bl_ref, wr_ref,
                 out_ref, dinv_ref):
    deg = jnp.sum(deg_ref[...], axis=0)
    dinv = 1.0 / jnp.maximum(deg, 1.0)
    dinv_ref[...] = dinv
    mean = (agg_ref[0] + agg_ref[1]) * dinv
    h = h_ref[...]
    z = (jnp.dot(mean, wl_ref[...], preferred_element_type=jnp.float32, precision=lax.Precision.HIGHEST)
         + bl_ref[...]
         + jnp.dot(h, wr_ref[...], preferred_element_type=jnp.float32, precision=lax.Precision.HIGHEST))
    out_ref[...] = jnp.maximum(z, 0.0) + h


_tc_layer1 = pl.pallas_call(
    _layer1_body,
    grid=(_NGRID,),
    in_specs=[
        pl.BlockSpec((_NC, _BLK, _H), lambda i: (0, i, 0)),
        pl.BlockSpec((_NW, _BLK, 1), lambda i: (0, i, 0)),
        pl.BlockSpec((_BLK, _H), lambda i: (i, 0)),
        pl.BlockSpec((_H, _H), lambda i: (0, 0)),
        pl.BlockSpec((1, _H), lambda i: (0, 0)),
        pl.BlockSpec((_H, _H), lambda i: (0, 0)),
    ],
    out_specs=[
        pl.BlockSpec((_BLK, _H), lambda i: (i, 0)),
        pl.BlockSpec((_BLK, 1), lambda i: (i, 0)),
    ],
    out_shape=[
        jax.ShapeDtypeStruct((_N, _H), jnp.float32),
        jax.ShapeDtypeStruct((_N, 1), jnp.float32),
    ],
)


def _layer_body(agg_ref, dinv_ref, h_ref, wl_ref, bl_ref, wr_ref, out_ref):
    mean = (agg_ref[0] + agg_ref[1]) * dinv_ref[...]
    h = h_ref[...]
    z = (jnp.dot(mean, wl_ref[...], preferred_element_type=jnp.float32, precision=lax.Precision.HIGHEST)
         + bl_ref[...]
         + jnp.dot(h, wr_ref[...], preferred_element_type=jnp.float32, precision=lax.Precision.HIGHEST))
    out_ref[...] = jnp.maximum(z, 0.0) + h


_tc_layer = pl.pallas_call(
    _layer_body,
    grid=(_NGRID,),
    in_specs=[
        pl.BlockSpec((_NC, _BLK, _H), lambda i: (0, i, 0)),
        pl.BlockSpec((_BLK, 1), lambda i: (i, 0)),
        pl.BlockSpec((_BLK, _H), lambda i: (i, 0)),
        pl.BlockSpec((_H, _H), lambda i: (0, 0)),
        pl.BlockSpec((1, _H), lambda i: (0, 0)),
        pl.BlockSpec((_H, _H), lambda i: (0, 0)),
    ],
    out_specs=pl.BlockSpec((_BLK, _H), lambda i: (i, 0)),
    out_shape=jax.ShapeDtypeStruct((_N, _H), jnp.float32),
)


def _final_body(agg_ref, dinv_ref, h_ref, wl_ref, bl_ref, wr_ref,
                batch_ref, wh_ref, bh_ref, out_ref, gsum_ref, gcnt_ref):
    i = pl.program_id(0)

    @pl.when(i == 0)
    def _():
        gsum_ref[...] = jnp.zeros_like(gsum_ref)
        gcnt_ref[...] = jnp.zeros_like(gcnt_ref)

    mean = (agg_ref[0] + agg_ref[1]) * dinv_ref[...]
    h = h_ref[...]
    z = (jnp.dot(mean, wl_ref[...], preferred_element_type=jnp.float32, precision=lax.Precision.HIGHEST)
         + bl_ref[...]
         + jnp.dot(h, wr_ref[...], preferred_element_type=jnp.float32, precision=lax.Precision.HIGHEST))
    h3 = jnp.maximum(z, 0.0) + h
    rowc = i * _BLK + lax.broadcasted_iota(jnp.int32, (_BLK, 1), 0)
    h3 = jnp.where(rowc < _N, h3, 0.0)

    row = i * _BLK + lax.broadcasted_iota(jnp.int32, (_BLK, _G), 0)
    valid = (row < _N).astype(jnp.float32)
    onehot = (batch_ref[...] ==
              lax.broadcasted_iota(jnp.int32, (_BLK, _G), 1)).astype(
                  jnp.float32) * valid
    gsum_ref[...] += lax.dot_general(
        onehot, h3, (((0,), (0,)), ((), ())),
        preferred_element_type=jnp.float32, precision=lax.Precision.HIGHEST)
    gcnt_ref[...] += lax.dot_general(
        onehot, jnp.ones((_BLK, 1), jnp.float32), (((0,), (0,)), ((), ())),
        preferred_element_type=jnp.float32, precision=lax.Precision.HIGHEST)

    @pl.when(i == _NGRID - 1)
    def _():
        g = gsum_ref[...] / jnp.maximum(gcnt_ref[...], 1.0)
        out_ref[...] = (
            jnp.dot(g, wh_ref[...], preferred_element_type=jnp.float32, precision=lax.Precision.HIGHEST)
            + bh_ref[...])


_tc_final = pl.pallas_call(
    _final_body,
    grid=(_NGRID,),
    in_specs=[
        pl.BlockSpec((_NC, _BLK, _H), lambda i: (0, i, 0)),
        pl.BlockSpec((_BLK, 1), lambda i: (i, 0)),
        pl.BlockSpec((_BLK, _H), lambda i: (i, 0)),
        pl.BlockSpec((_H, _H), lambda i: (0, 0)),
        pl.BlockSpec((1, _H), lambda i: (0, 0)),
        pl.BlockSpec((_H, _H), lambda i: (0, 0)),
        pl.BlockSpec((_BLK, 1), lambda i: (i, 0)),
        pl.BlockSpec((_H, _C), lambda i: (0, 0)),
        pl.BlockSpec((1, _C), lambda i: (0, 0)),
    ],
    out_specs=pl.BlockSpec((_G, _C), lambda i: (0, 0)),
    out_shape=jax.ShapeDtypeStruct((_G, _C), jnp.float32),
    scratch_shapes=[
        pltpu.VMEM((_G, _H), jnp.float32),
        pltpu.VMEM((_G, 1), jnp.float32),
    ],
)


def kernel(x, edge_index, batch, W_l1, b_l1, W_r1, W_l2, b_l2, W_r2,
           W_l3, b_l3, W_r3, W_head, b_head):
    src = edge_index[0]
    dst = edge_index[1]
    npad = _EPAD - _E
    srcp = jnp.concatenate(
        [src, jnp.zeros((npad,), jnp.int32)]).reshape(_NW, _CPW, _CHUNK)
    dstp = jnp.concatenate(
        [dst, jnp.full((npad,), _TRASH, jnp.int32)]).reshape(_NW, _CPW, _CHUNK)
    zrow = jnp.zeros((_NPAD, _H), jnp.float32)
    zflat = jnp.zeros((_NBINS,), jnp.float32)
    batch2 = batch.reshape(_N, 1)

    agg1, deg1 = _seg1(x, srcp, dstp, zrow, zflat)
    deg1r = deg1.reshape(_NW, _NBINS, 1)
    h1, dinv = _tc_layer1(agg1, deg1r, x,
                          W_l1.T, b_l1.reshape(1, _H), W_r1.T)
    agg2 = _seg(h1, srcp, dstp, zrow)
    h2 = _tc_layer(agg2, dinv, h1, W_l2.T, b_l2.reshape(1, _H), W_r2.T)
    agg3 = _seg(h2, srcp, dstp, zrow)
    return _tc_final(agg3, dinv, h2, W_l3.T, b_l3.reshape(1, _H), W_r3.T,
                     batch2, W_head.T, b_head.reshape(1, _C))
